# baseline (speedup 1.0000x reference)
import jax
import jax.numpy as jnp
import numpy as np
from jax import lax
from jax.experimental import pallas as pl
from jax.experimental.pallas import tpu as pltpu

_ALPHA = 0.4
_B = 4096
_R = 128
_LBLK = ((0, 32), (32, 32), (64, 32), (96, 32), (128, 1))
_SBLK = ((0, 32), (32, 32), (64, 32), (96, 32))
_INTERPRET = False


def _host_perm():
    key = jax.random.key(1)
    kp, _ = jax.random.split(key)
    return np.asarray(jax.random.permutation(kp, _B))


def _schedule(perm, R):
    B = perm.shape[0]
    visited = np.zeros(B, bool)
    segs = []
    for s in range(B):
        if visited[s]:
            continue
        c = []
        j = s
        while not visited[j]:
            visited[j] = True
            c.append(int(j))
            j = int(perm[j])
        m = len(c)
        p = R - (m % R) if m % R else R
        segs.append([c[t % m] for t in range(m + p)])
    T = [x for seg in segs for x in seg]
    M = len(T)
    nsteps = M // R
    seg_ends = set(np.cumsum([len(s) for s in segs]).tolist())
    winv = [-1 if (i * R + R) in seg_ends else T[i * R + R - 1]
            for i in range(nsteps)]
    return (np.asarray(T + [0], np.int32), np.asarray(winv, np.int32),
            np.asarray(T, np.int32), nsteps)


_PERM_NP = _host_perm()
_T_EXT, _WINV, _T, _NSTEPS = _schedule(_PERM_NP, _R)


def _body(t_ref, winv_ref, lam_ref, hp_ref, hq_ref, op_ref, oq_ref,
          inp, inq, obp, obq, dust, lsp, lsq, ssp, ssq):
    i = pl.program_id(0)
    slot = lax.rem(i, 2)

    def issue_loads(step, s):
        base = step * _R
        for k, (st, ln) in enumerate(_LBLK):
            for j in range(st, st + ln):
                row = t_ref[base + j]
                pltpu.make_async_copy(
                    hp_ref.at[pl.ds(row, 1)], inp.at[s, pl.ds(j, 1)],
                    lsp.at[s, k]).start()
                pltpu.make_async_copy(
                    hq_ref.at[pl.ds(row, 1)], inq.at[s, pl.ds(j, 1)],
                    lsq.at[s, k]).start()

    def drain_loads(s):
        for k, (st, ln) in enumerate(_LBLK):
            pltpu.make_async_copy(
                hp_ref.at[pl.ds(0, ln)], inp.at[s, pl.ds(st, ln)],
                lsp.at[s, k]).wait()
            pltpu.make_async_copy(
                hq_ref.at[pl.ds(0, ln)], inq.at[s, pl.ds(st, ln)],
                lsq.at[s, k]).wait()

    def issue_stores(s):
        base = i * _R
        for k, (st, ln) in enumerate(_SBLK):
            for j in range(st, st + ln):
                if j < _R - 1:
                    row = t_ref[base + j]
                    pltpu.make_async_copy(
                        obp.at[s, pl.ds(j, 1)], op_ref.at[pl.ds(row, 1)],
                        ssp.at[s, k]).start()
                    pltpu.make_async_copy(
                        obq.at[s, pl.ds(j, 1)], oq_ref.at[pl.ds(row, 1)],
                        ssq.at[s, k]).start()
                else:
                    wv = winv_ref[i]

                    @pl.when(wv >= 0)
                    def _():
                        pltpu.make_async_copy(
                            obp.at[s, pl.ds(j, 1)],
                            op_ref.at[pl.ds(jnp.maximum(wv, 0), 1)],
                            ssp.at[s, k]).start()
                        pltpu.make_async_copy(
                            obq.at[s, pl.ds(j, 1)],
                            oq_ref.at[pl.ds(jnp.maximum(wv, 0), 1)],
                            ssq.at[s, k]).start()

                    @pl.when(wv < 0)
                    def _():
                        pltpu.make_async_copy(
                            obp.at[s, pl.ds(j, 1)], dust.at[pl.ds(0, 1)],
                            ssp.at[s, k]).start()
                        pltpu.make_async_copy(
                            obq.at[s, pl.ds(j, 1)], dust.at[pl.ds(1, 1)],
                            ssq.at[s, k]).start()

    def drain_stores(s):
        for k, (st, ln) in enumerate(_SBLK):
            pltpu.make_async_copy(
                obp.at[s, pl.ds(st, ln)], op_ref.at[pl.ds(0, ln)],
                ssp.at[s, k]).wait()
            pltpu.make_async_copy(
                obq.at[s, pl.ds(st, ln)], oq_ref.at[pl.ds(0, ln)],
                ssq.at[s, k]).wait()

    @pl.when(i == 0)
    def _():
        issue_loads(0, 0)
        issue_loads(1, 1)

    drain_loads(slot)

    @pl.when(i >= 2)
    def _():
        drain_stores(slot)

    lam = lam_ref[...]
    one_m = 1.0 - lam
    rp = inp.at[slot]
    rq = inq.at[slot]
    a = rp[pl.ds(0, _R), :]
    bb = rp[pl.ds(1, _R), :]
    obp.at[slot][...] = a + jnp.log(lam + one_m * jnp.exp(bb - a))
    a = rq[pl.ds(0, _R), :]
    bb = rq[pl.ds(1, _R), :]
    obq.at[slot][...] = a + jnp.log(lam + one_m * jnp.exp(bb - a))

    issue_stores(slot)

    @pl.when(i + 2 < _NSTEPS)
    def _():
        issue_loads(i + 2, slot)

    @pl.when(i == _NSTEPS - 1)
    def _():
        drain_stores(slot)
        drain_stores(lax.rem(i + 1, 2))


def kernel(x_pre, x_post):
    b, g = x_pre.shape
    key = jax.random.key(1)
    kp, kl = jax.random.split(key)
    perm = jax.random.permutation(kp, b)
    lam = jax.random.beta(kl, _ALPHA, _ALPHA, (b,)).astype(jnp.float32)

    t_sched = jnp.asarray(_T)
    lam_sched = lam[t_sched].reshape(-1, 1)

    hbm = pl.BlockSpec(memory_space=pl.ANY)
    lam_spec = pl.BlockSpec((_R, 1), lambda i, t, w: (i, 0))

    grid_spec = pltpu.PrefetchScalarGridSpec(
        num_scalar_prefetch=2,
        grid=(_NSTEPS,),
        in_specs=[lam_spec, hbm, hbm],
        out_specs=[hbm, hbm],
        scratch_shapes=[
            pltpu.VMEM((2, _R + 1, g), jnp.float32),
            pltpu.VMEM((2, _R + 1, g), jnp.float32),
            pltpu.VMEM((2, _R, g), jnp.float32),
            pltpu.VMEM((2, _R, g), jnp.float32),
            pltpu.VMEM((8, g), jnp.float32),
            pltpu.SemaphoreType.DMA((2, 5)),
            pltpu.SemaphoreType.DMA((2, 5)),
            pltpu.SemaphoreType.DMA((2, 5)),
            pltpu.SemaphoreType.DMA((2, 5)),
        ],
    )
    out_shape = [jax.ShapeDtypeStruct((b, g), jnp.float32)] * 2
    op, oq = pl.pallas_call(
        _body,
        grid_spec=grid_spec,
        out_shape=out_shape,
        interpret=_INTERPRET,
    )(jnp.asarray(_T_EXT), jnp.asarray(_WINV), lam_sched, x_pre, x_post)
    return op, oq, lam, perm


# speedup vs baseline: 1.2146x; 1.2146x over previous
import jax
import jax.numpy as jnp
from jax import lax
from jax.experimental import pallas as pl
from jax.experimental.pallas import tpu as pltpu

_ALPHA = 0.4
_R = 128
_NBUF = 2
_INTERPRET = False


def _mix_body(perm_ref, ap_ref, aq_ref, lam_ref, hp_ref, hq_ref,
              op_ref, oq_ref, bufs_p, bufs_q, semp, semq):
    i = pl.program_id(0)
    n = pl.num_programs(0)
    rem = lax.rem(i, _NBUF)

    def issue(step, s):
        base = step * _R
        for j in range(_R):
            row = perm_ref[base + j]
            pltpu.make_async_copy(
                hp_ref.at[pl.ds(row, 1)], bufs_p[s].at[pl.ds(j, 1)],
                semp.at[s, j % 8]).start()
            pltpu.make_async_copy(
                hq_ref.at[pl.ds(row, 1)], bufs_q[s].at[pl.ds(j, 1)],
                semq.at[s, j % 8]).start()

    @pl.when(i == 0)
    def _():
        for k in range(_NBUF - 1):
            issue(k, k)

    lam = lam_ref[...]
    one_m = 1.0 - lam

    def step_for(s):
        s2 = (s + _NBUF - 1) % _NBUF

        @pl.when(i + _NBUF - 1 < n)
        def _():
            issue(i + _NBUF - 1, s2)

        for j in range(_R):
            pltpu.make_async_copy(
                hp_ref.at[pl.ds(0, 1)], bufs_p[s].at[pl.ds(j, 1)],
                semp.at[s, j % 8]).wait()
            pltpu.make_async_copy(
                hq_ref.at[pl.ds(0, 1)], bufs_q[s].at[pl.ds(j, 1)],
                semq.at[s, j % 8]).wait()
        a = ap_ref[...]
        b = bufs_p[s][...]
        op_ref[...] = a + jnp.log(lam + one_m * jnp.exp(b - a))
        a = aq_ref[...]
        b = bufs_q[s][...]
        oq_ref[...] = a + jnp.log(lam + one_m * jnp.exp(b - a))

    for s in range(_NBUF):
        @pl.when(rem == s)
        def _(s=s):
            step_for(s)


def kernel(x_pre, x_post):
    b, g = x_pre.shape
    key = jax.random.key(1)
    kp, kl = jax.random.split(key)
    perm = jax.random.permutation(kp, b)
    lam = jax.random.beta(kl, _ALPHA, _ALPHA, (b,)).astype(jnp.float32)

    hbm = pl.BlockSpec(memory_space=pl.ANY)
    lam_spec = pl.BlockSpec((_R, 1), lambda i, perm_r: (i, 0))
    out_spec = pl.BlockSpec((_R, g), lambda i, perm_r: (i, 0))

    grid_spec = pltpu.PrefetchScalarGridSpec(
        num_scalar_prefetch=1,
        grid=(b // _R,),
        in_specs=[out_spec, out_spec, lam_spec, hbm, hbm],
        out_specs=[out_spec, out_spec],
        scratch_shapes=[
            [pltpu.VMEM((_R, g), jnp.float32) for _ in range(_NBUF)],
            [pltpu.VMEM((_R, g), jnp.float32) for _ in range(_NBUF)],
            pltpu.SemaphoreType.DMA((_NBUF, 8)),
            pltpu.SemaphoreType.DMA((_NBUF, 8)),
        ],
    )
    out_shape = [jax.ShapeDtypeStruct((b, g), jnp.float32)] * 2
    op, oq = pl.pallas_call(
        _mix_body,
        grid_spec=grid_spec,
        out_shape=out_shape,
        interpret=_INTERPRET,
    )(perm, x_pre, x_post, lam.reshape(b, 1), x_pre, x_post)
    return op, oq, lam, perm


# speedup vs baseline: 1.2170x; 1.0020x over previous
import jax
import jax.numpy as jnp
from jax import lax
from jax.experimental import pallas as pl
from jax.experimental.pallas import tpu as pltpu

_ALPHA = 0.4
_R = 128
_NBUF = 2
_INTERPRET = False


def _mix_body(perm_ref, ap_ref, aq_ref, lam_ref, hp_ref, hq_ref,
              op_ref, oq_ref, bufs_p, bufs_q, semp, semq):
    i = pl.program_id(0)
    n = pl.num_programs(0)
    rem = lax.rem(i, _NBUF)

    def issue(step, s):
        base = step * _R
        for j in range(_R):
            row = perm_ref[base + j]
            pltpu.make_async_copy(
                hp_ref.at[pl.ds(row, 1)], bufs_p[s].at[pl.ds(j, 1)],
                semp.at[s, j % 4]).start()
            pltpu.make_async_copy(
                hq_ref.at[pl.ds(row, 1)], bufs_q[s].at[pl.ds(j, 1)],
                semq.at[s, j % 4]).start()

    @pl.when(i == 0)
    def _():
        for k in range(_NBUF - 1):
            issue(k, k)

    lam = lam_ref[...]
    one_m = 1.0 - lam

    def step_for(s):
        s2 = (s + _NBUF - 1) % _NBUF

        @pl.when(i + _NBUF - 1 < n)
        def _():
            issue(i + _NBUF - 1, s2)

        for j in range(_R):
            pltpu.make_async_copy(
                hp_ref.at[pl.ds(0, 1)], bufs_p[s].at[pl.ds(j, 1)],
                semp.at[s, j % 4]).wait()
            pltpu.make_async_copy(
                hq_ref.at[pl.ds(0, 1)], bufs_q[s].at[pl.ds(j, 1)],
                semq.at[s, j % 4]).wait()
        a = ap_ref[...]
        b = bufs_p[s][...]
        op_ref[...] = a + jnp.log(lam + one_m * jnp.exp(b - a))
        a = aq_ref[...]
        b = bufs_q[s][...]
        oq_ref[...] = a + jnp.log(lam + one_m * jnp.exp(b - a))

    for s in range(_NBUF):
        @pl.when(rem == s)
        def _(s=s):
            step_for(s)


def kernel(x_pre, x_post):
    b, g = x_pre.shape
    key = jax.random.key(1)
    kp, kl = jax.random.split(key)
    perm = jax.random.permutation(kp, b)
    lam = jax.random.beta(kl, _ALPHA, _ALPHA, (b,)).astype(jnp.float32)

    hbm = pl.BlockSpec(memory_space=pl.ANY)
    lam_spec = pl.BlockSpec((_R, 1), lambda i, perm_r: (i, 0))
    out_spec = pl.BlockSpec((_R, g), lambda i, perm_r: (i, 0))

    grid_spec = pltpu.PrefetchScalarGridSpec(
        num_scalar_prefetch=1,
        grid=(b // _R,),
        in_specs=[out_spec, out_spec, lam_spec, hbm, hbm],
        out_specs=[out_spec, out_spec],
        scratch_shapes=[
            [pltpu.VMEM((_R, g), jnp.float32) for _ in range(_NBUF)],
            [pltpu.VMEM((_R, g), jnp.float32) for _ in range(_NBUF)],
            pltpu.SemaphoreType.DMA((_NBUF, 4)),
            pltpu.SemaphoreType.DMA((_NBUF, 4)),
        ],
    )
    out_shape = [jax.ShapeDtypeStruct((b, g), jnp.float32)] * 2
    op, oq = pl.pallas_call(
        _mix_body,
        grid_spec=grid_spec,
        out_shape=out_shape,
        interpret=_INTERPRET,
    )(perm, x_pre, x_post, lam.reshape(b, 1), x_pre, x_post)
    return op, oq, lam, perm


# speedup vs baseline: 1.2404x; 1.0192x over previous
import jax
import jax.numpy as jnp
from jax import lax
from jax.experimental import pallas as pl
from jax.experimental.pallas import tpu as pltpu

_ALPHA = 0.4
_R = 64
_NBUF = 3
_INTERPRET = False


def _mix_body(perm_ref, ap_ref, aq_ref, lam_ref, hp_ref, hq_ref,
              op_ref, oq_ref, bufs_p, bufs_q, semp, semq):
    i = pl.program_id(0)
    n = pl.num_programs(0)
    rem = lax.rem(i, _NBUF)

    def issue(step, s):
        base = step * _R
        for j in range(_R):
            row = perm_ref[base + j]
            pltpu.make_async_copy(
                hp_ref.at[pl.ds(row, 1)], bufs_p[s].at[pl.ds(j, 1)],
                semp.at[s, j % 4]).start()
            pltpu.make_async_copy(
                hq_ref.at[pl.ds(row, 1)], bufs_q[s].at[pl.ds(j, 1)],
                semq.at[s, j % 4]).start()

    @pl.when(i == 0)
    def _():
        for k in range(_NBUF - 1):
            issue(k, k)

    lam = lam_ref[...]
    one_m = 1.0 - lam

    def step_for(s):
        s2 = (s + _NBUF - 1) % _NBUF

        @pl.when(i + _NBUF - 1 < n)
        def _():
            issue(i + _NBUF - 1, s2)

        for j in range(_R):
            pltpu.make_async_copy(
                hp_ref.at[pl.ds(0, 1)], bufs_p[s].at[pl.ds(j, 1)],
                semp.at[s, j % 4]).wait()
            pltpu.make_async_copy(
                hq_ref.at[pl.ds(0, 1)], bufs_q[s].at[pl.ds(j, 1)],
                semq.at[s, j % 4]).wait()
        a = ap_ref[...]
        b = bufs_p[s][...]
        op_ref[...] = a + jnp.log(lam + one_m * jnp.exp(b - a))
        a = aq_ref[...]
        b = bufs_q[s][...]
        oq_ref[...] = a + jnp.log(lam + one_m * jnp.exp(b - a))

    for s in range(_NBUF):
        @pl.when(rem == s)
        def _(s=s):
            step_for(s)


def kernel(x_pre, x_post):
    b, g = x_pre.shape
    key = jax.random.key(1)
    kp, kl = jax.random.split(key)
    perm = jax.random.permutation(kp, b)
    lam = jax.random.beta(kl, _ALPHA, _ALPHA, (b,)).astype(jnp.float32)

    hbm = pl.BlockSpec(memory_space=pl.ANY)
    lam_spec = pl.BlockSpec((_R, 1), lambda i, perm_r: (i, 0))
    out_spec = pl.BlockSpec((_R, g), lambda i, perm_r: (i, 0))

    grid_spec = pltpu.PrefetchScalarGridSpec(
        num_scalar_prefetch=1,
        grid=(b // _R,),
        in_specs=[out_spec, out_spec, lam_spec, hbm, hbm],
        out_specs=[out_spec, out_spec],
        scratch_shapes=[
            [pltpu.VMEM((_R, g), jnp.float32) for _ in range(_NBUF)],
            [pltpu.VMEM((_R, g), jnp.float32) for _ in range(_NBUF)],
            pltpu.SemaphoreType.DMA((_NBUF, 4)),
            pltpu.SemaphoreType.DMA((_NBUF, 4)),
        ],
    )
    out_shape = [jax.ShapeDtypeStruct((b, g), jnp.float32)] * 2
    op, oq = pl.pallas_call(
        _mix_body,
        grid_spec=grid_spec,
        out_shape=out_shape,
        interpret=_INTERPRET,
    )(perm, x_pre, x_post, lam.reshape(b, 1), x_pre, x_post)
    return op, oq, lam, perm


# speedup vs baseline: 1.2432x; 1.0022x over previous
import jax
import jax.numpy as jnp
from jax import lax
from jax.experimental import pallas as pl
from jax.experimental.pallas import tpu as pltpu

_ALPHA = 0.4
_R = 64
_NBUF = 3


def _mix_body(perm_ref, ap_ref, aq_ref, lam_ref, hp_ref, hq_ref,
              op_ref, oq_ref, bufs_p, bufs_q, semp, semq):
    i = pl.program_id(0)
    n = pl.num_programs(0)
    rem = lax.rem(i, _NBUF)

    def issue(step, s):
        base = step * _R
        for j in range(_R):
            row = perm_ref[base + j]
            pltpu.make_async_copy(
                hp_ref.at[pl.ds(row, 1)], bufs_p[s].at[pl.ds(j, 1)],
                semp.at[s, j % 4]).start()
            pltpu.make_async_copy(
                hq_ref.at[pl.ds(row, 1)], bufs_q[s].at[pl.ds(j, 1)],
                semq.at[s, j % 4]).start()

    @pl.when(i == 0)
    def _():
        for k in range(_NBUF - 1):
            issue(k, k)

    lam = lam_ref[...]
    one_m = 1.0 - lam

    def step_for(s):
        s2 = (s + _NBUF - 1) % _NBUF

        @pl.when(i + _NBUF - 1 < n)
        def _():
            issue(i + _NBUF - 1, s2)

        for j in range(_R):
            pltpu.make_async_copy(
                hp_ref.at[pl.ds(0, 1)], bufs_p[s].at[pl.ds(j, 1)],
                semp.at[s, j % 4]).wait()
            pltpu.make_async_copy(
                hq_ref.at[pl.ds(0, 1)], bufs_q[s].at[pl.ds(j, 1)],
                semq.at[s, j % 4]).wait()
        a = ap_ref[...]
        b = bufs_p[s][...]
        op_ref[...] = a + jnp.log(lam + one_m * jnp.exp(b - a))
        a = aq_ref[...]
        b = bufs_q[s][...]
        oq_ref[...] = a + jnp.log(lam + one_m * jnp.exp(b - a))

    for s in range(_NBUF):
        @pl.when(rem == s)
        def _(s=s):
            step_for(s)


def kernel(x_pre, x_post):
    b, g = x_pre.shape
    key = jax.random.key(1)
    kp, kl = jax.random.split(key)
    perm = jax.random.permutation(kp, b)
    lam = jax.random.beta(kl, _ALPHA, _ALPHA, (b,)).astype(jnp.float32)

    hbm = pl.BlockSpec(memory_space=pl.ANY)
    lam_spec = pl.BlockSpec((_R, 1), lambda i, perm_r: (i, 0))
    out_spec = pl.BlockSpec((_R, g), lambda i, perm_r: (i, 0))

    grid_spec = pltpu.PrefetchScalarGridSpec(
        num_scalar_prefetch=1,
        grid=(b // _R,),
        in_specs=[out_spec, out_spec, lam_spec, hbm, hbm],
        out_specs=[out_spec, out_spec],
        scratch_shapes=[
            [pltpu.VMEM((_R, g), jnp.float32) for _ in range(_NBUF)],
            [pltpu.VMEM((_R, g), jnp.float32) for _ in range(_NBUF)],
            pltpu.SemaphoreType.DMA((_NBUF, 4)),
            pltpu.SemaphoreType.DMA((_NBUF, 4)),
        ],
    )
    out_shape = [jax.ShapeDtypeStruct((b, g), jnp.float32)] * 2
    op, oq = pl.pallas_call(
        _mix_body,
        grid_spec=grid_spec,
        out_shape=out_shape,
    )(perm, x_pre, x_post, lam.reshape(b, 1), x_pre, x_post)
    return op, oq, lam, perm
